# baseline (speedup 1.0000x reference)
import functools

import jax
import jax.numpy as jnp
from jax import lax
from jax.experimental import pallas as pl
from jax.experimental.pallas import tpu as pltpu
from jax.experimental.pallas import tpu_sc as plsc

NUM_V = 1000000
D = 64
B = 16384
L = 16
CHUNK = 128
UNROLL = 16


def _build():
    info = plsc.get_sparse_core_info()
    nc, ns = info.num_cores, info.num_subcores
    nw = nc * ns
    bpw = B // nw
    nround = bpw // CHUNK

    mesh = plsc.VectorSubcoreMesh(core_axis_name="c", subcore_axis_name="s")

    @functools.partial(
        pl.kernel,
        mesh=mesh,
        compiler_params=pltpu.CompilerParams(
            needs_layout_passes=False, use_tc_tiling_on_sc=False),
        out_type=jax.ShapeDtypeStruct((B,), jnp.float32),
        scratch_types=[
            pltpu.VMEM((nround, CHUNK), jnp.int32),
            pltpu.VMEM((nround, CHUNK), jnp.int32),
            pltpu.VMEM((2, CHUNK, D), jnp.float32),
            pltpu.VMEM((2, CHUNK, D), jnp.float32),
            pltpu.VMEM((2, CHUNK, D), jnp.float32),
            pltpu.VMEM((2, CHUNK, D), jnp.float32),
            pltpu.VMEM((bpw,), jnp.float32),
            pltpu.SemaphoreType.DMA,
            pltpu.SemaphoreType.DMA,
        ],
    )
    def line_kernel(u_i2, u_j2, wf, ws, wc, out,
                    idx_i, idx_j, a1, b1, a2, b2, out_v, sem0, sem1):
        wid = lax.axis_index("s") * nc + lax.axis_index("c")
        rowbase = wid * nround
        pltpu.sync_copy(u_i2.at[pl.ds(rowbase, nround)], idx_i)
        pltpu.sync_copy(u_j2.at[pl.ds(rowbase, nround)], idx_j)
        sems = (sem0, sem1)

        def fire(k):
            s = k % 2
            ik = idx_i.at[k]
            jk = idx_j.at[k]
            return (
                pltpu.async_copy(wf.at[ik], a1.at[s], sems[s]),
                pltpu.async_copy(wf.at[jk], b1.at[s], sems[s]),
                pltpu.async_copy(ws.at[ik], a2.at[s], sems[s]),
                pltpu.async_copy(wc.at[jk], b2.at[s], sems[s]),
            )

        def compute(k):
            s = k % 2
            A1, B1, A2, B2 = a1.at[s], b1.at[s], a2.at[s], b2.at[s]
            lanes = lax.iota(jnp.int32, L)

            def body(i, carry):
                row_v = i * L + lanes
                acc = jnp.zeros((L,), jnp.float32)
                acc2 = jnp.zeros((L,), jnp.float32)
                for c in range(D):
                    col_v = jnp.full((L,), c, jnp.int32)
                    acc = acc + (plsc.load_gather(A1, [row_v, col_v])
                                 * plsc.load_gather(B1, [row_v, col_v]))
                    acc2 = acc2 + (plsc.load_gather(A2, [row_v, col_v])
                                   * plsc.load_gather(B2, [row_v, col_v]))
                out_v[pl.ds(k * CHUNK + i * L, L)] = acc + acc2
                return carry

            lax.fori_loop(0, CHUNK // L, body, 0)

        pend = fire(0)
        for k in range(nround):
            for d in pend:
                d.wait()
            if k + 1 < nround:
                pend = fire(k + 1)
            compute(k)

        pltpu.sync_copy(out_v, out.at[pl.ds(wid * bpw, bpw)])

    return line_kernel


def kernel(u_i, u_j, W_first, W_second, W_context):
    u_i2 = u_i.astype(jnp.int32).reshape(B // CHUNK, CHUNK)
    u_j2 = u_j.astype(jnp.int32).reshape(B // CHUNK, CHUNK)
    return _build()(u_i2, u_j2, W_first, W_second, W_context)


# speedup vs baseline: 1.2152x; 1.2152x over previous
import functools

import jax
import jax.numpy as jnp
from jax import lax
from jax.experimental import pallas as pl
from jax.experimental.pallas import tpu as pltpu
from jax.experimental.pallas import tpu_sc as plsc

NUM_V = 1000000
D = 64
B = 16384
L = 16
CHUNK = 64
TBLK = 4096


def _tc_transpose_pairs(wt):
    grid = (NUM_V + TBLK - 1) // TBLK

    def body(in_ref, o_ref):
        xt = in_ref[...].T.reshape(TBLK // 2, 2, D)
        o_ref[:, :D] = xt[:, 0, :]
        o_ref[:, D:] = xt[:, 1, :]

    return pl.pallas_call(
        body,
        grid=(grid,),
        in_specs=[pl.BlockSpec((D, TBLK), lambda i: (0, i))],
        out_specs=pl.BlockSpec((TBLK // 2, 2 * D), lambda i: (i, 0)),
        out_shape=jax.ShapeDtypeStruct((NUM_V // 2, 2 * D), jnp.float32),
    )(wt)


def _build():
    info = plsc.get_sparse_core_info()
    nc, ns = info.num_cores, info.num_subcores
    nw = nc * ns
    bpw = B // nw
    nround = bpw // CHUNK

    mesh = plsc.VectorSubcoreMesh(core_axis_name="c", subcore_axis_name="s")

    @functools.partial(
        pl.kernel,
        mesh=mesh,
        compiler_params=pltpu.CompilerParams(needs_layout_passes=False),
        out_type=jax.ShapeDtypeStruct((B,), jnp.float32),
        scratch_types=[
            pltpu.VMEM((nround, CHUNK), jnp.int32),
            pltpu.VMEM((nround, CHUNK), jnp.int32),
            pltpu.VMEM((nround, CHUNK), jnp.int32),
            pltpu.VMEM((nround, CHUNK), jnp.int32),
            pltpu.VMEM((2, CHUNK, 2 * D), jnp.float32),
            pltpu.VMEM((2, CHUNK, 2 * D), jnp.float32),
            pltpu.VMEM((2, CHUNK, 2 * D), jnp.float32),
            pltpu.VMEM((2, CHUNK, 2 * D), jnp.float32),
            pltpu.VMEM((bpw,), jnp.float32),
            pltpu.SemaphoreType.DMA,
            pltpu.SemaphoreType.DMA,
        ],
    )
    def line_kernel(ui_h2, ui_p2, uj_h2, uj_p2, wf2, ws2, wc2, out,
                    idx_ih, idx_ip, idx_jh, idx_jp,
                    a1, b1, a2, b2, out_v, sem0, sem1):
        wid = lax.axis_index("s") * nc + lax.axis_index("c")
        rowbase = wid * nround
        pltpu.sync_copy(ui_h2.at[pl.ds(rowbase, nround)], idx_ih)
        pltpu.sync_copy(ui_p2.at[pl.ds(rowbase, nround)], idx_ip)
        pltpu.sync_copy(uj_h2.at[pl.ds(rowbase, nround)], idx_jh)
        pltpu.sync_copy(uj_p2.at[pl.ds(rowbase, nround)], idx_jp)
        sems = (sem0, sem1)

        def fire(k, s):
            ik = idx_ih.at[k]
            jk = idx_jh.at[k]
            return (
                pltpu.async_copy(wf2.at[ik], a1.at[s], sems[s]),
                pltpu.async_copy(wf2.at[jk], b1.at[s], sems[s]),
                pltpu.async_copy(ws2.at[ik], a2.at[s], sems[s]),
                pltpu.async_copy(wc2.at[jk], b2.at[s], sems[s]),
            )

        def compute(k, s):
            A1, B1, A2, B2 = a1.at[s], b1.at[s], a2.at[s], b2.at[s]
            lanes = lax.iota(jnp.int32, L)

            def body(i, carry):
                row_v = i * L + lanes
                pi = idx_ip[k, pl.ds(i * L, L)]
                pj = idx_jp[k, pl.ds(i * L, L)]
                acc = jnp.zeros((L,), jnp.float32)
                acc2 = jnp.zeros((L,), jnp.float32)
                for c in range(D):
                    ci = pi + c
                    cj = pj + c
                    acc = acc + (plsc.load_gather(A1, [row_v, ci])
                                 * plsc.load_gather(B1, [row_v, cj]))
                    acc2 = acc2 + (plsc.load_gather(A2, [row_v, ci])
                                   * plsc.load_gather(B2, [row_v, cj]))
                out_v[pl.ds(k * CHUNK + i * L, L)] = acc + acc2
                return carry

            lax.fori_loop(0, CHUNK // L, body, 0)

        pend = fire(0, 0)
        for k in range(nround):
            for d in pend:
                d.wait()
            if k + 1 < nround:
                pend = fire(k + 1, (k + 1) % 2)
            compute(k, k % 2)

        pltpu.sync_copy(out_v, out.at[pl.ds(wid * bpw, bpw)])

    return line_kernel


def kernel(u_i, u_j, W_first, W_second, W_context):
    shape2 = (B // CHUNK, CHUNK)
    u_i = u_i.astype(jnp.int32)
    u_j = u_j.astype(jnp.int32)
    ui_h2 = (u_i >> 1).reshape(shape2)
    ui_p2 = ((u_i & 1) << 6).reshape(shape2)
    uj_h2 = (u_j >> 1).reshape(shape2)
    uj_p2 = ((u_j & 1) << 6).reshape(shape2)
    wf2 = _tc_transpose_pairs(W_first.T)
    ws2 = W_second.reshape(NUM_V // 2, 2 * D)
    wc2 = W_context.reshape(NUM_V // 2, 2 * D)
    return _build()(ui_h2, ui_p2, uj_h2, uj_p2, wf2, ws2, wc2)


# speedup vs baseline: 1.9387x; 1.5953x over previous
import functools

import jax
import jax.numpy as jnp
from jax import lax
from jax.experimental import pallas as pl
from jax.experimental.pallas import tpu as pltpu
from jax.experimental.pallas import tpu_sc as plsc

NUM_V = 1000000
D = 64
B = 16384
L = 16
CHUNK = 64
TBLK = 8192


def _tc_transpose_pairs(wt):
    grid = (NUM_V + TBLK - 1) // TBLK
    nq = grid * (TBLK // 2)

    def body(in_ref, o_ref):
        x = in_ref[...]
        eye = (lax.broadcasted_iota(jnp.int32, (D, D), 0)
               == lax.broadcasted_iota(jnp.int32, (D, D), 1)).astype(jnp.float32)
        xt = lax.dot_general(x, eye, (((0,), (0,)), ((), ())),
                             preferred_element_type=jnp.float32)
        o_ref[:, :D] = xt[:TBLK // 2, :]
        o_ref[:, D:] = xt[TBLK // 2:, :]

    return pl.pallas_call(
        body,
        grid=(grid,),
        in_specs=[pl.BlockSpec((D, TBLK), lambda i: (0, i))],
        out_specs=pl.BlockSpec((TBLK // 2, 2 * D), lambda i: (i, 0)),
        out_shape=jax.ShapeDtypeStruct((nq, 2 * D), jnp.float32),
    )(wt)


def _build():
    info = plsc.get_sparse_core_info()
    nc, ns = info.num_cores, info.num_subcores
    nw = nc * ns
    bpw = B // nw
    nround = bpw // CHUNK

    mesh = plsc.VectorSubcoreMesh(core_axis_name="c", subcore_axis_name="s")

    @functools.partial(
        pl.kernel,
        mesh=mesh,
        compiler_params=pltpu.CompilerParams(needs_layout_passes=False),
        out_type=jax.ShapeDtypeStruct((B,), jnp.float32),
        scratch_types=[
            pltpu.VMEM((nround, CHUNK), jnp.int32),
            pltpu.VMEM((nround, CHUNK), jnp.int32),
            pltpu.VMEM((nround, CHUNK), jnp.int32),
            pltpu.VMEM((nround, CHUNK), jnp.int32),
            pltpu.VMEM((2, CHUNK, 2 * D), jnp.float32),
            pltpu.VMEM((2, CHUNK, 2 * D), jnp.float32),
            pltpu.VMEM((2, CHUNK, 2 * D), jnp.float32),
            pltpu.VMEM((2, CHUNK, 2 * D), jnp.float32),
            pltpu.VMEM((bpw,), jnp.float32),
            pltpu.SemaphoreType.DMA,
            pltpu.SemaphoreType.DMA,
        ],
    )
    def line_kernel(ui_h2, ui_p2, uj_h2, uj_p2, wf2, ws2, wc2, out,
                    idx_ih, idx_ip, idx_jh, idx_jp,
                    a1, b1, a2, b2, out_v, sem0, sem1):
        wid = lax.axis_index("s") * nc + lax.axis_index("c")
        rowbase = wid * nround
        pltpu.sync_copy(ui_h2.at[pl.ds(rowbase, nround)], idx_ih)
        pltpu.sync_copy(ui_p2.at[pl.ds(rowbase, nround)], idx_ip)
        pltpu.sync_copy(uj_h2.at[pl.ds(rowbase, nround)], idx_jh)
        pltpu.sync_copy(uj_p2.at[pl.ds(rowbase, nround)], idx_jp)
        sems = (sem0, sem1)

        def fire(k, s):
            ik = idx_ih.at[k]
            jk = idx_jh.at[k]
            return (
                pltpu.async_copy(wf2.at[ik], a1.at[s], sems[s]),
                pltpu.async_copy(wf2.at[jk], b1.at[s], sems[s]),
                pltpu.async_copy(ws2.at[ik], a2.at[s], sems[s]),
                pltpu.async_copy(wc2.at[jk], b2.at[s], sems[s]),
            )

        def compute(k, s):
            A1, B1, A2, B2 = a1.at[s], b1.at[s], a2.at[s], b2.at[s]
            lanes = lax.iota(jnp.int32, L)

            def body(i, carry):
                row_v = i * L + lanes
                pi = idx_ip[k, pl.ds(i * L, L)]
                pj = idx_jp[k, pl.ds(i * L, L)]
                acc = jnp.zeros((L,), jnp.float32)
                acc2 = jnp.zeros((L,), jnp.float32)
                for c in range(D):
                    ci = pi + c
                    cj = pj + c
                    acc = acc + (plsc.load_gather(A1, [row_v, ci])
                                 * plsc.load_gather(B1, [row_v, cj]))
                    acc2 = acc2 + (plsc.load_gather(A2, [row_v, ci])
                                   * plsc.load_gather(B2, [row_v, cj]))
                out_v[pl.ds(k * CHUNK + i * L, L)] = acc + acc2
                return carry

            lax.fori_loop(0, CHUNK // L, body, 0)

        pend = fire(0, 0)
        for k in range(nround):
            for d in pend:
                d.wait()
            if k + 1 < nround:
                pend = fire(k + 1, (k + 1) % 2)
            compute(k, k % 2)

        pltpu.sync_copy(out_v, out.at[pl.ds(wid * bpw, bpw)])

    return line_kernel


def kernel(u_i, u_j, W_first, W_second, W_context):
    shape2 = (B // CHUNK, CHUNK)
    u_i = u_i.astype(jnp.int32)
    u_j = u_j.astype(jnp.int32)
    def q_of(v):
        return (((v >> 13) << 12) | (v & 4095)).reshape(shape2)

    def p_of(v):
        return (((v >> 12) & 1) << 6).reshape(shape2)

    ui_h2, ui_p2 = q_of(u_i), p_of(u_i)
    uj_h2, uj_p2 = q_of(u_j), p_of(u_j)
    wf2 = _tc_transpose_pairs(W_first.T)
    ws2 = _tc_transpose_pairs(W_second.T)
    wc2 = _tc_transpose_pairs(W_context.T)
    return _build()(ui_h2, ui_p2, uj_h2, uj_p2, wf2, ws2, wc2)


# speedup vs baseline: 2.3778x; 1.2265x over previous
import functools

import jax
import jax.numpy as jnp
from jax import lax
from jax.experimental import pallas as pl
from jax.experimental.pallas import tpu as pltpu
from jax.experimental.pallas import tpu_sc as plsc

NUM_V = 1000000
D = 64
B = 16384
L = 16
CHUNK = 64
TBLK = 8192


def _tc_transpose_pairs(wt):
    grid = (NUM_V + TBLK - 1) // TBLK
    nq = grid * (TBLK // 2)

    def body(f_ref, s_ref, c_ref, of_ref, os_ref, oc_ref):
        for in_ref, o_ref in ((f_ref, of_ref), (s_ref, os_ref), (c_ref, oc_ref)):
            xt = in_ref[...].T
            o_ref[:, :D] = xt[:TBLK // 2, :]
            o_ref[:, D:] = xt[TBLK // 2:, :]

    spec_in = pl.BlockSpec((D, TBLK), lambda i: (0, i))
    spec_out = pl.BlockSpec((TBLK // 2, 2 * D), lambda i: (i, 0))
    shape_out = jax.ShapeDtypeStruct((nq, 2 * D), jnp.float32)
    return pl.pallas_call(
        body,
        grid=(grid,),
        in_specs=[spec_in, spec_in, spec_in],
        out_specs=[spec_out, spec_out, spec_out],
        out_shape=[shape_out, shape_out, shape_out],
    )(*wt)


def _build():
    info = plsc.get_sparse_core_info()
    nc, ns = info.num_cores, info.num_subcores
    nw = nc * ns
    bpw = B // nw
    nround = bpw // CHUNK

    mesh = plsc.VectorSubcoreMesh(core_axis_name="c", subcore_axis_name="s")

    @functools.partial(
        pl.kernel,
        mesh=mesh,
        compiler_params=pltpu.CompilerParams(needs_layout_passes=False),
        out_type=jax.ShapeDtypeStruct((B,), jnp.float32),
        scratch_types=[
            pltpu.VMEM((nround, CHUNK), jnp.int32),
            pltpu.VMEM((nround, CHUNK), jnp.int32),
            pltpu.VMEM((nround, CHUNK), jnp.int32),
            pltpu.VMEM((nround, CHUNK), jnp.int32),
            pltpu.VMEM((2, CHUNK, 2 * D), jnp.float32),
            pltpu.VMEM((2, CHUNK, 2 * D), jnp.float32),
            pltpu.VMEM((2, CHUNK, 2 * D), jnp.float32),
            pltpu.VMEM((2, CHUNK, 2 * D), jnp.float32),
            pltpu.VMEM((bpw,), jnp.float32),
            pltpu.SemaphoreType.DMA,
            pltpu.SemaphoreType.DMA,
        ],
    )
    def line_kernel(ui_h2, ui_p2, uj_h2, uj_p2, wf2, ws2, wc2, out,
                    idx_ih, idx_ip, idx_jh, idx_jp,
                    a1, b1, a2, b2, out_v, sem0, sem1):
        wid = lax.axis_index("s") * nc + lax.axis_index("c")
        rowbase = wid * nround
        pltpu.sync_copy(ui_h2.at[pl.ds(rowbase, nround)], idx_ih)
        pltpu.sync_copy(ui_p2.at[pl.ds(rowbase, nround)], idx_ip)
        pltpu.sync_copy(uj_h2.at[pl.ds(rowbase, nround)], idx_jh)
        pltpu.sync_copy(uj_p2.at[pl.ds(rowbase, nround)], idx_jp)
        sems = (sem0, sem1)

        def fire(k, s):
            ik = idx_ih.at[k]
            jk = idx_jh.at[k]
            return (
                pltpu.async_copy(wf2.at[ik], a1.at[s], sems[s]),
                pltpu.async_copy(wf2.at[jk], b1.at[s], sems[s]),
                pltpu.async_copy(ws2.at[ik], a2.at[s], sems[s]),
                pltpu.async_copy(wc2.at[jk], b2.at[s], sems[s]),
            )

        def compute(k, s):
            A1, B1, A2, B2 = a1.at[s], b1.at[s], a2.at[s], b2.at[s]
            lanes = lax.iota(jnp.int32, L)

            def body(i, carry):
                row_v = i * L + lanes
                pi = idx_ip[k, pl.ds(i * L, L)]
                pj = idx_jp[k, pl.ds(i * L, L)]
                acc = jnp.zeros((L,), jnp.float32)
                acc2 = jnp.zeros((L,), jnp.float32)
                for c in range(D):
                    ci = pi + c
                    cj = pj + c
                    acc = acc + (plsc.load_gather(A1, [row_v, ci])
                                 * plsc.load_gather(B1, [row_v, cj]))
                    acc2 = acc2 + (plsc.load_gather(A2, [row_v, ci])
                                   * plsc.load_gather(B2, [row_v, cj]))
                out_v[pl.ds(k * CHUNK + i * L, L)] = acc + acc2
                return carry

            lax.fori_loop(0, CHUNK // L, body, 0)

        pend = fire(0, 0)
        for k in range(nround):
            for d in pend:
                d.wait()
            if k + 1 < nround:
                pend = fire(k + 1, (k + 1) % 2)
            compute(k, k % 2)

        pltpu.sync_copy(out_v, out.at[pl.ds(wid * bpw, bpw)])

    return line_kernel


def kernel(u_i, u_j, W_first, W_second, W_context):
    shape2 = (B // CHUNK, CHUNK)
    u_i = u_i.astype(jnp.int32)
    u_j = u_j.astype(jnp.int32)
    def q_of(v):
        return (((v >> 13) << 12) | (v & 4095)).reshape(shape2)

    def p_of(v):
        return (((v >> 12) & 1) << 6).reshape(shape2)

    ui_h2, ui_p2 = q_of(u_i), p_of(u_i)
    uj_h2, uj_p2 = q_of(u_j), p_of(u_j)
    wf2, ws2, wc2 = _tc_transpose_pairs(
        (W_first.T, W_second.T, W_context.T))
    return _build()(ui_h2, ui_p2, uj_h2, uj_p2, wf2, ws2, wc2)
